# baseline (speedup 1.0000x reference)
import functools

import jax
import jax.numpy as jnp
from jax import lax
from jax.experimental import pallas as pl
from jax.experimental.pallas import tpu as pltpu
from jax.experimental.pallas import tpu_sc as plsc

F32 = jnp.float32
I32 = jnp.int32

_D_IN = 1024
_D_MODEL = 256
_D_OUT = 64
_NL = 2
_E = 8
_DS = 16
_DC = 4
_DI = 512
_DTR = 16
_DFF = 1024
_T = 2048

_TB = 256
_TC = 128
_NP = 2 * _T
_SB = 128
_NBLK = _NP // _SB + _E
_NSLOT = _NBLK * _SB


def _ln_in(x, g, b, eps=1e-5):
    m = jnp.mean(x, axis=-1, keepdims=True)
    v = jnp.mean((x - m) * (x - m), axis=-1, keepdims=True)
    return (x - m) * lax.rsqrt(v + eps) * g + b


def _silu(x):
    return x * jax.nn.sigmoid(x)


def _dot(a, b):
    return jnp.dot(a, b, preferred_element_type=F32)


def _kin_body(z_ref, w_ref, b_ref, o_ref):
    o_ref[...] = _dot(z_ref[...], w_ref[...]) + b_ref[...]


def _input_proj(z2d, W_in, b_in):
    nb = _T // _TB
    return pl.pallas_call(
        _kin_body,
        grid=(nb,),
        in_specs=[
            pl.BlockSpec((_TB, _D_IN), lambda i: (i, 0)),
            pl.BlockSpec((_D_IN, _D_MODEL), lambda i: (0, 0)),
            pl.BlockSpec((1, _D_MODEL), lambda i: (0, 0)),
        ],
        out_specs=pl.BlockSpec((_TB, _D_MODEL), lambda i: (i, 0)),
        out_shape=jax.ShapeDtypeStruct((_T, _D_MODEL), F32),
    )(z2d, W_in, b_in.reshape(1, -1))


def _kfront_body(x_ref, g_ref, b_ref, wi_ref, cw_ref, cb_ref, wx_ref,
                 wdt_ref, bdt_ref, xc_ref, z_ref, dt_ref, bc_ref, carry_ref):
    i = pl.program_id(0)

    @pl.when(i == 0)
    def _():
        carry_ref[...] = jnp.zeros_like(carry_ref)

    xn = _ln_in(x_ref[...], g_ref[...], b_ref[...])
    xz = _dot(xn, wi_ref[...])
    xs = xz[:, :_DI]
    zb = xz[:, _DI:]
    carry = carry_ref[0:_DC - 1]
    seg = jnp.concatenate([carry, xs], axis=0)
    conv = cb_ref[...]
    for j in range(_DC):
        conv = conv + seg[j:j + _TB] * cw_ref[j:j + 1, :]
    carry_ref[0:_DC - 1] = xs[_TB - (_DC - 1):_TB]
    xc = _silu(conv)
    xc_ref[...] = xc
    z_ref[...] = zb
    xdbl = _dot(xc, wx_ref[...])
    bc_ref[...] = xdbl[:, _DTR:]
    dt_ref[...] = jax.nn.softplus(_dot(xdbl[:, :_DTR], wdt_ref[...])
                                  + bdt_ref[...])


def _mamba_front(x, mg, mb, Wi, cwT, cb, Wx, Wdt, bdt):
    nb = _T // _TB
    outs = (
        jax.ShapeDtypeStruct((_T, _DI), F32),
        jax.ShapeDtypeStruct((_T, _DI), F32),
        jax.ShapeDtypeStruct((_T, _DI), F32),
        jax.ShapeDtypeStruct((_T, 2 * _DS), F32),
    )
    return pl.pallas_call(
        _kfront_body,
        grid=(nb,),
        in_specs=[
            pl.BlockSpec((_TB, _D_MODEL), lambda i: (i, 0)),
            pl.BlockSpec((1, _D_MODEL), lambda i: (0, 0)),
            pl.BlockSpec((1, _D_MODEL), lambda i: (0, 0)),
            pl.BlockSpec((_D_MODEL, 2 * _DI), lambda i: (0, 0)),
            pl.BlockSpec((_DC, _DI), lambda i: (0, 0)),
            pl.BlockSpec((1, _DI), lambda i: (0, 0)),
            pl.BlockSpec((_DI, _DTR + 2 * _DS), lambda i: (0, 0)),
            pl.BlockSpec((_DTR, _DI), lambda i: (0, 0)),
            pl.BlockSpec((1, _DI), lambda i: (0, 0)),
        ],
        out_specs=(
            pl.BlockSpec((_TB, _DI), lambda i: (i, 0)),
            pl.BlockSpec((_TB, _DI), lambda i: (i, 0)),
            pl.BlockSpec((_TB, _DI), lambda i: (i, 0)),
            pl.BlockSpec((_TB, 2 * _DS), lambda i: (i, 0)),
        ),
        out_shape=outs,
        scratch_shapes=[pltpu.VMEM((8, _DI), F32)],
    )(x, mg.reshape(1, -1), mb.reshape(1, -1), Wi, cwT, cb.reshape(1, -1),
      Wx, Wdt, bdt.reshape(1, -1))


def _kscan_body(xc_ref, z_ref, dt_ref, bc_ref, negA_ref, dp_ref, wo_ref,
                r_ref, o_ref, A_ref, B_ref, h_ref):
    i = pl.program_id(0)

    @pl.when(i == 0)
    def _():
        h_ref[...] = jnp.zeros_like(h_ref)

    dt_c = dt_ref[...]
    xc_c = xc_ref[...]
    u = dt_c * xc_c
    B_c = bc_ref[:, :_DS]
    C_c = bc_ref[:, _DS:]
    A_ref[...] = jnp.exp(dt_c[:, None, :] * negA_ref[...][None, :, :])
    B_ref[...] = B_c[:, :, None] * u[:, None, :]

    k = 1
    while k < _TC:
        Pa = A_ref[...]
        Ba = B_ref[...]
        A_ref[k:] = Pa[k:] * Pa[:_TC - k]
        B_ref[k:] = Ba[k:] + Pa[k:] * Ba[:_TC - k]
        k *= 2

    h0 = h_ref[...]
    H = A_ref[...] * h0[None, :, :] + B_ref[...]
    h_ref[...] = H[_TC - 1]
    Y = jnp.sum(H * C_c[:, :, None], axis=1)
    y = (Y + dp_ref[...] * xc_c) * _silu(z_ref[...])
    o_ref[...] = r_ref[...] + _dot(y, wo_ref[...])


def _mamba_scan(xc, z, dt, bc, negA_T, Dp, Wo, r):
    nb = _T // _TC
    return pl.pallas_call(
        _kscan_body,
        grid=(nb,),
        in_specs=[
            pl.BlockSpec((_TC, _DI), lambda i: (i, 0)),
            pl.BlockSpec((_TC, _DI), lambda i: (i, 0)),
            pl.BlockSpec((_TC, _DI), lambda i: (i, 0)),
            pl.BlockSpec((_TC, 2 * _DS), lambda i: (i, 0)),
            pl.BlockSpec((_DS, _DI), lambda i: (0, 0)),
            pl.BlockSpec((1, _DI), lambda i: (0, 0)),
            pl.BlockSpec((_DI, _D_MODEL), lambda i: (0, 0)),
            pl.BlockSpec((_TC, _D_MODEL), lambda i: (i, 0)),
        ],
        out_specs=pl.BlockSpec((_TC, _D_MODEL), lambda i: (i, 0)),
        out_shape=jax.ShapeDtypeStruct((_T, _D_MODEL), F32),
        scratch_shapes=[
            pltpu.VMEM((_TC, _DS, _DI), F32),
            pltpu.VMEM((_TC, _DS, _DI), F32),
            pltpu.VMEM((_DS, _DI), F32),
        ],
    )(xc, z, dt, bc, negA_T, Dp.reshape(1, -1), Wo, r)


def _krouter_body(x_ref, g_ref, b_ref, wr_ref, xn_ref, e1_ref, e2_ref,
                  g1_ref, g2_ref):
    xn = _ln_in(x_ref[...], g_ref[...], b_ref[...])
    xn_ref[...] = xn
    logits = _dot(xn, wr_ref[...])
    mx = jnp.max(logits, axis=-1, keepdims=True)
    ex = jnp.exp(logits - mx)
    probs = ex / jnp.sum(ex, axis=-1, keepdims=True)
    lane = lax.broadcasted_iota(I32, probs.shape, 1)
    i1 = jnp.argmax(probs, axis=-1)[:, None]
    p1 = jnp.max(probs, axis=-1, keepdims=True)
    masked = jnp.where(lane == i1, -jnp.inf, probs)
    i2 = jnp.argmax(masked, axis=-1)[:, None]
    p2 = jnp.max(masked, axis=-1, keepdims=True)
    denom = p1 + p2
    e1_ref[...] = i1
    e2_ref[...] = i2
    g1_ref[...] = p1 / denom
    g2_ref[...] = p2 / denom


def _router(x, og, ob, Wr):
    nb = _T // _TB
    return pl.pallas_call(
        _krouter_body,
        grid=(nb,),
        in_specs=[
            pl.BlockSpec((_TB, _D_MODEL), lambda i: (i, 0)),
            pl.BlockSpec((1, _D_MODEL), lambda i: (0, 0)),
            pl.BlockSpec((1, _D_MODEL), lambda i: (0, 0)),
            pl.BlockSpec((_D_MODEL, _E), lambda i: (0, 0)),
        ],
        out_specs=(
            pl.BlockSpec((_TB, _D_MODEL), lambda i: (i, 0)),
            pl.BlockSpec((_TB, 1), lambda i: (i, 0)),
            pl.BlockSpec((_TB, 1), lambda i: (i, 0)),
            pl.BlockSpec((_TB, 1), lambda i: (i, 0)),
            pl.BlockSpec((_TB, 1), lambda i: (i, 0)),
        ),
        out_shape=(
            jax.ShapeDtypeStruct((_T, _D_MODEL), F32),
            jax.ShapeDtypeStruct((_T, 1), I32),
            jax.ShapeDtypeStruct((_T, 1), I32),
            jax.ShapeDtypeStruct((_T, 1), F32),
            jax.ShapeDtypeStruct((_T, 1), F32),
        ),
    )(x, og.reshape(1, -1), ob.reshape(1, -1), Wr)


def _kplan_body(e1_ref, e2_ref, dst_ref, eid_ref):
    lane1 = lax.broadcasted_iota(I32, (_T, _E), 1)
    x1 = (lane1 == e1_ref[...]).astype(F32)
    x2 = (lane1 == e2_ref[...]).astype(F32)
    cnt = (jnp.sum(x1, axis=0, keepdims=True)
           + jnp.sum(x2, axis=0, keepdims=True))
    padcnt = jnp.ceil(cnt / _SB) * _SB
    tri8 = (lax.broadcasted_iota(I32, (_E, _E), 0)
            < lax.broadcasted_iota(I32, (_E, _E), 1)).astype(F32)
    pad_off = _dot(padcnt, tri8)

    pbo = jnp.transpose(pad_off, (1, 0)) / _SB
    bidx = lax.broadcasted_iota(I32, (_E, _NBLK), 1).astype(F32)
    eid_ref[...] = (jnp.sum((bidx >= pbo).astype(I32), axis=0,
                            keepdims=True) - 1)

    tri = (lax.broadcasted_iota(I32, (_SB, _SB), 0)
           >= lax.broadcasted_iota(I32, (_SB, _SB), 1)).astype(F32)

    def block(b, x, carry):
        xb = x[b * _SB:(b + 1) * _SB, :]
        s = _dot(tri, xb)
        rank = s - xb + carry
        dst = jnp.sum(xb * (rank + pad_off), axis=1, keepdims=True)
        return dst.astype(I32), carry + s[_SB - 1:_SB, :]

    carry = jnp.zeros((1, _E), F32)
    for b in range(_T // _SB):
        dst, carry = block(b, x1, carry)
        dst_ref[pl.ds(b * _SB, _SB), :] = dst
    for b in range(_T // _SB):
        dst, carry = block(b, x2, carry)
        dst_ref[pl.ds(_T + b * _SB, _SB), :] = dst


def _plan(e1, e2):
    return pl.pallas_call(
        _kplan_body,
        grid=(1,),
        in_specs=[
            pl.BlockSpec((_T, 1), lambda i: (0, 0)),
            pl.BlockSpec((_T, 1), lambda i: (0, 0)),
        ],
        out_specs=(
            pl.BlockSpec((_NP, 1), lambda i: (0, 0)),
            pl.BlockSpec((1, _NBLK), lambda i: (0, 0)),
        ),
        out_shape=(
            jax.ShapeDtypeStruct((_NP, 1), I32),
            jax.ShapeDtypeStruct((1, _NBLK), I32),
        ),
    )(e1, e2)


def _sc_dispatch(xn, srctok, dst):
    info = plsc.get_sparse_core_info()
    nw = info.num_cores * info.num_subcores
    bpw = _NP // nw
    mesh = plsc.VectorSubcoreMesh(core_axis_name="c", subcore_axis_name="s")

    @functools.partial(
        pl.kernel, mesh=mesh,
        out_type=jax.ShapeDtypeStruct((_NSLOT, _D_MODEL), F32),
        scratch_types=[
            pltpu.VMEM((bpw,), I32),
            pltpu.VMEM((bpw,), I32),
            pltpu.VMEM((bpw, _D_MODEL), F32),
            pltpu.SemaphoreType.DMA,
            pltpu.SemaphoreType.DMA,
        ],
    )
    def k(xn_hbm, src_hbm, dst_hbm, xg_hbm, idxs_v, idxd_v, rows_v, s1, s2):
        wid = lax.axis_index("s") * info.num_cores + lax.axis_index("c")
        base = wid * bpw
        pltpu.sync_copy(src_hbm.at[pl.ds(base, bpw)], idxs_v)
        pltpu.sync_copy(dst_hbm.at[pl.ds(base, bpw)], idxd_v)
        pltpu.async_copy(xn_hbm.at[idxs_v], rows_v, s1).wait()
        pltpu.async_copy(rows_v, xg_hbm.at[idxd_v], s2).wait()

    return k(xn, srctok, dst)


def _sc_combine_gather(og, dst):
    info = plsc.get_sparse_core_info()
    nw = info.num_cores * info.num_subcores
    bpw = _NP // nw
    mesh = plsc.VectorSubcoreMesh(core_axis_name="c", subcore_axis_name="s")

    @functools.partial(
        pl.kernel, mesh=mesh,
        out_type=jax.ShapeDtypeStruct((_NP, _D_MODEL), F32),
        scratch_types=[
            pltpu.VMEM((bpw,), I32),
            pltpu.VMEM((bpw, _D_MODEL), F32),
            pltpu.SemaphoreType.DMA,
        ],
    )
    def k(og_hbm, dst_hbm, y2_hbm, idx_v, rows_v, sem):
        wid = lax.axis_index("s") * info.num_cores + lax.axis_index("c")
        base = wid * bpw
        pltpu.sync_copy(dst_hbm.at[pl.ds(base, bpw)], idx_v)
        pltpu.async_copy(og_hbm.at[idx_v], rows_v, sem).wait()
        pltpu.sync_copy(rows_v, y2_hbm.at[pl.ds(base, bpw)])

    return k(og, dst)


def _kgffn_body(eid_ref, xg_ref, w1_ref, b1_ref, w2_ref, b2_ref, og_ref):
    hpre = _dot(xg_ref[...], w1_ref[0]) + b1_ref[0]
    h = 0.5 * hpre * (1.0 + lax.erf(hpre * 0.7071067811865476))
    og_ref[...] = _dot(h, w2_ref[0]) + b2_ref[0]


def _grouped_ffn(eid, xg, W1, b1, W2, b2):
    grid_spec = pltpu.PrefetchScalarGridSpec(
        num_scalar_prefetch=1,
        grid=(_NBLK,),
        in_specs=[
            pl.BlockSpec((_SB, _D_MODEL), lambda b, eid_ref: (b, 0)),
            pl.BlockSpec((1, _D_MODEL, _DFF),
                         lambda b, eid_ref: (eid_ref[b], 0, 0)),
            pl.BlockSpec((1, 1, _DFF), lambda b, eid_ref: (eid_ref[b], 0, 0)),
            pl.BlockSpec((1, _DFF, _D_MODEL),
                         lambda b, eid_ref: (eid_ref[b], 0, 0)),
            pl.BlockSpec((1, 1, _D_MODEL),
                         lambda b, eid_ref: (eid_ref[b], 0, 0)),
        ],
        out_specs=pl.BlockSpec((_SB, _D_MODEL), lambda b, eid_ref: (b, 0)),
    )
    return pl.pallas_call(
        _kgffn_body,
        grid_spec=grid_spec,
        out_shape=jax.ShapeDtypeStruct((_NSLOT, _D_MODEL), F32),
    )(eid, xg, W1, b1.reshape(_E, 1, _DFF), W2, b2.reshape(_E, 1, _D_MODEL))


def _kcomb_body(r_ref, ya_ref, yb_ref, g1_ref, g2_ref, o_ref):
    o_ref[...] = (r_ref[...] + g1_ref[...] * ya_ref[...]
                  + g2_ref[...] * yb_ref[...])


def _combine(r, y2, g1, g2):
    nb = _T // _TB
    return pl.pallas_call(
        _kcomb_body,
        grid=(nb,),
        in_specs=[
            pl.BlockSpec((_TB, _D_MODEL), lambda i: (i, 0)),
            pl.BlockSpec((_TB, _D_MODEL), lambda i: (i, 0)),
            pl.BlockSpec((_TB, _D_MODEL), lambda i: (i + _T // _TB, 0)),
            pl.BlockSpec((_TB, 1), lambda i: (i, 0)),
            pl.BlockSpec((_TB, 1), lambda i: (i, 0)),
        ],
        out_specs=pl.BlockSpec((_TB, _D_MODEL), lambda i: (i, 0)),
        out_shape=jax.ShapeDtypeStruct((_T, _D_MODEL), F32),
    )(r, y2, y2, g1, g2)


def _moe_layer(x, og_, ob_, Wr, W1, b1, W2, b2, srctok):
    xn, e1, e2, g1, g2 = _router(x, og_, ob_, Wr)
    dst, eid = _plan(e1, e2)
    dst1 = dst.reshape(_NP)
    xg = _sc_dispatch(xn, srctok, dst1)
    og = _grouped_ffn(eid.reshape(_NBLK), xg, W1, b1, W2, b2)
    y2 = _sc_combine_gather(og, dst1)
    return _combine(x, y2, g1, g2)


def _kout_body(x_ref, w_ref, b_ref, g_ref, bb_ref, o_ref):
    o = _dot(x_ref[...], w_ref[...]) + b_ref[...]
    o_ref[...] = _ln_in(o, g_ref[...], bb_ref[...])


def _out_proj(x, W_out, b_out, ln_g, ln_b):
    nb = _T // _TB
    return pl.pallas_call(
        _kout_body,
        grid=(nb,),
        in_specs=[
            pl.BlockSpec((_TB, _D_MODEL), lambda i: (i, 0)),
            pl.BlockSpec((_D_MODEL, _D_OUT), lambda i: (0, 0)),
            pl.BlockSpec((1, _D_OUT), lambda i: (0, 0)),
            pl.BlockSpec((1, _D_OUT), lambda i: (0, 0)),
            pl.BlockSpec((1, _D_OUT), lambda i: (0, 0)),
        ],
        out_specs=pl.BlockSpec((_TB, _D_OUT), lambda i: (i, 0)),
        out_shape=jax.ShapeDtypeStruct((_T, _D_OUT), F32),
    )(x, W_out, b_out.reshape(1, -1), ln_g.reshape(1, -1),
      ln_b.reshape(1, -1))


def kernel(z_t, W_in, b_in, in_proj_W, conv_w, conv_b, x_proj_W, dt_proj_W,
           dt_proj_b, A_log, D_param, out_proj_W, router_W, eW1, eb1, eW2,
           eb2, mn_g, mn_b, on_g, on_b, W_out, b_out, ln_g, ln_b):
    z2d = z_t.reshape(_T, _D_IN)
    srctok = jnp.concatenate([jnp.arange(_T, dtype=I32)] * 2)
    x = _input_proj(z2d, W_in, b_in)
    for i in range(_NL):
        cwT = jnp.transpose(conv_w[i], (1, 0))
        negA_T = -jnp.exp(jnp.transpose(A_log[i], (1, 0)))
        xc, z, dt, bc = _mamba_front(x, mn_g[i], mn_b[i], in_proj_W[i], cwT,
                                     conv_b[i], x_proj_W[i], dt_proj_W[i],
                                     dt_proj_b[i])
        x = _mamba_scan(xc, z, dt, bc, negA_T, D_param[i], out_proj_W[i], x)
        x = _moe_layer(x, on_g[i], on_b[i], router_W[i], eW1[i], eb1[i],
                       eW2[i], eb2[i], srctok)
    out = _out_proj(x, W_out, b_out, ln_g, ln_b)
    return out.reshape(1, _T, _D_OUT)


# speedup vs baseline: 1.2900x; 1.2900x over previous
import functools

import jax
import jax.numpy as jnp
from jax import lax
from jax.experimental import pallas as pl
from jax.experimental.pallas import tpu as pltpu
from jax.experimental.pallas import tpu_sc as plsc

F32 = jnp.float32
I32 = jnp.int32

_D_IN = 1024
_D_MODEL = 256
_D_OUT = 64
_NL = 2
_E = 8
_DS = 16
_DC = 4
_DI = 512
_DTR = 16
_DFF = 1024
_T = 2048

_TB = 256
_TC = 128
_NP = 2 * _T
_SB = 128
_NBLK = _NP // _SB + _E
_NSLOT = _NBLK * _SB


def _ln_in(x, g, b, eps=1e-5):
    m = jnp.mean(x, axis=-1, keepdims=True)
    v = jnp.mean((x - m) * (x - m), axis=-1, keepdims=True)
    return (x - m) * lax.rsqrt(v + eps) * g + b


def _silu(x):
    return x * jax.nn.sigmoid(x)


def _dot(a, b):
    return jnp.dot(a, b, preferred_element_type=F32)


def _kin_body(z_ref, w_ref, b_ref, o_ref):
    o_ref[...] = _dot(z_ref[...], w_ref[...]) + b_ref[...]


def _input_proj(z2d, W_in, b_in):
    nb = _T // _TB
    return pl.pallas_call(
        _kin_body,
        grid=(nb,),
        in_specs=[
            pl.BlockSpec((_TB, _D_IN), lambda i: (i, 0)),
            pl.BlockSpec((_D_IN, _D_MODEL), lambda i: (0, 0)),
            pl.BlockSpec((1, _D_MODEL), lambda i: (0, 0)),
        ],
        out_specs=pl.BlockSpec((_TB, _D_MODEL), lambda i: (i, 0)),
        out_shape=jax.ShapeDtypeStruct((_T, _D_MODEL), F32),
    )(z2d, W_in, b_in.reshape(1, -1))


def _kfront_body(x_ref, g_ref, b_ref, wi_ref, cw_ref, cb_ref, wx_ref,
                 wdt_ref, bdt_ref, xc_ref, z_ref, dt_ref, bc_ref, carry_ref):
    i = pl.program_id(0)

    @pl.when(i == 0)
    def _():
        carry_ref[...] = jnp.zeros_like(carry_ref)

    xn = _ln_in(x_ref[...], g_ref[...], b_ref[...])
    xz = _dot(xn, wi_ref[...])
    xs = xz[:, :_DI]
    zb = xz[:, _DI:]
    carry = carry_ref[0:_DC - 1]
    seg = jnp.concatenate([carry, xs], axis=0)
    conv = cb_ref[...]
    for j in range(_DC):
        conv = conv + seg[j:j + _TB] * cw_ref[j:j + 1, :]
    carry_ref[0:_DC - 1] = xs[_TB - (_DC - 1):_TB]
    xc = _silu(conv)
    xc_ref[...] = xc
    z_ref[...] = zb
    xdbl = _dot(xc, wx_ref[...])
    bc_ref[...] = xdbl[:, _DTR:]
    dt_ref[...] = jax.nn.softplus(_dot(xdbl[:, :_DTR], wdt_ref[...])
                                  + bdt_ref[...])


def _mamba_front(x, mg, mb, Wi, cwT, cb, Wx, Wdt, bdt):
    nb = _T // _TB
    outs = (
        jax.ShapeDtypeStruct((_T, _DI), F32),
        jax.ShapeDtypeStruct((_T, _DI), F32),
        jax.ShapeDtypeStruct((_T, _DI), F32),
        jax.ShapeDtypeStruct((_T, 2 * _DS), F32),
    )
    return pl.pallas_call(
        _kfront_body,
        grid=(nb,),
        in_specs=[
            pl.BlockSpec((_TB, _D_MODEL), lambda i: (i, 0)),
            pl.BlockSpec((1, _D_MODEL), lambda i: (0, 0)),
            pl.BlockSpec((1, _D_MODEL), lambda i: (0, 0)),
            pl.BlockSpec((_D_MODEL, 2 * _DI), lambda i: (0, 0)),
            pl.BlockSpec((_DC, _DI), lambda i: (0, 0)),
            pl.BlockSpec((1, _DI), lambda i: (0, 0)),
            pl.BlockSpec((_DI, _DTR + 2 * _DS), lambda i: (0, 0)),
            pl.BlockSpec((_DTR, _DI), lambda i: (0, 0)),
            pl.BlockSpec((1, _DI), lambda i: (0, 0)),
        ],
        out_specs=(
            pl.BlockSpec((_TB, _DI), lambda i: (i, 0)),
            pl.BlockSpec((_TB, _DI), lambda i: (i, 0)),
            pl.BlockSpec((_TB, _DI), lambda i: (i, 0)),
            pl.BlockSpec((_TB, 2 * _DS), lambda i: (i, 0)),
        ),
        out_shape=outs,
        scratch_shapes=[pltpu.VMEM((8, _DI), F32)],
    )(x, mg.reshape(1, -1), mb.reshape(1, -1), Wi, cwT, cb.reshape(1, -1),
      Wx, Wdt, bdt.reshape(1, -1))


def _kscan_body(xc_ref, z_ref, dt_ref, bc_ref, negA_ref, dp_ref, wo_ref,
                r_ref, o_ref, A_ref, B_ref, h_ref):
    i = pl.program_id(0)

    @pl.when(i == 0)
    def _():
        h_ref[...] = jnp.zeros_like(h_ref)

    dt_c = dt_ref[...]
    xc_c = xc_ref[...]
    u = dt_c * xc_c
    B_c = bc_ref[:, :_DS]
    C_c = bc_ref[:, _DS:]
    A_ref[...] = jnp.exp(dt_c[:, None, :] * negA_ref[...][None, :, :])
    B_ref[...] = B_c[:, :, None] * u[:, None, :]

    h = h_ref[...]
    for t in range(_TC):
        h = A_ref[t] * h + B_ref[t]
        B_ref[t] = h
    h_ref[...] = h
    Y = jnp.sum(B_ref[...] * C_c[:, :, None], axis=1)
    y = (Y + dp_ref[...] * xc_c) * _silu(z_ref[...])
    o_ref[...] = r_ref[...] + _dot(y, wo_ref[...])


def _mamba_scan(xc, z, dt, bc, negA_T, Dp, Wo, r):
    nb = _T // _TC
    return pl.pallas_call(
        _kscan_body,
        grid=(nb,),
        in_specs=[
            pl.BlockSpec((_TC, _DI), lambda i: (i, 0)),
            pl.BlockSpec((_TC, _DI), lambda i: (i, 0)),
            pl.BlockSpec((_TC, _DI), lambda i: (i, 0)),
            pl.BlockSpec((_TC, 2 * _DS), lambda i: (i, 0)),
            pl.BlockSpec((_DS, _DI), lambda i: (0, 0)),
            pl.BlockSpec((1, _DI), lambda i: (0, 0)),
            pl.BlockSpec((_DI, _D_MODEL), lambda i: (0, 0)),
            pl.BlockSpec((_TC, _D_MODEL), lambda i: (i, 0)),
        ],
        out_specs=pl.BlockSpec((_TC, _D_MODEL), lambda i: (i, 0)),
        out_shape=jax.ShapeDtypeStruct((_T, _D_MODEL), F32),
        scratch_shapes=[
            pltpu.VMEM((_TC, _DS, _DI), F32),
            pltpu.VMEM((_TC, _DS, _DI), F32),
            pltpu.VMEM((_DS, _DI), F32),
        ],
    )(xc, z, dt, bc, negA_T, Dp.reshape(1, -1), Wo, r)


def _krouter_body(x_ref, g_ref, b_ref, wr_ref, xn_ref, e1_ref, e2_ref,
                  g1_ref, g2_ref):
    xn = _ln_in(x_ref[...], g_ref[...], b_ref[...])
    xn_ref[...] = xn
    logits = _dot(xn, wr_ref[...])
    mx = jnp.max(logits, axis=-1, keepdims=True)
    ex = jnp.exp(logits - mx)
    probs = ex / jnp.sum(ex, axis=-1, keepdims=True)
    lane = lax.broadcasted_iota(I32, probs.shape, 1)
    i1 = jnp.argmax(probs, axis=-1)[:, None]
    p1 = jnp.max(probs, axis=-1, keepdims=True)
    masked = jnp.where(lane == i1, -jnp.inf, probs)
    i2 = jnp.argmax(masked, axis=-1)[:, None]
    p2 = jnp.max(masked, axis=-1, keepdims=True)
    denom = p1 + p2
    e1_ref[...] = i1
    e2_ref[...] = i2
    g1_ref[...] = p1 / denom
    g2_ref[...] = p2 / denom


def _router(x, og, ob, Wr):
    nb = _T // _TB
    return pl.pallas_call(
        _krouter_body,
        grid=(nb,),
        in_specs=[
            pl.BlockSpec((_TB, _D_MODEL), lambda i: (i, 0)),
            pl.BlockSpec((1, _D_MODEL), lambda i: (0, 0)),
            pl.BlockSpec((1, _D_MODEL), lambda i: (0, 0)),
            pl.BlockSpec((_D_MODEL, _E), lambda i: (0, 0)),
        ],
        out_specs=(
            pl.BlockSpec((_TB, _D_MODEL), lambda i: (i, 0)),
            pl.BlockSpec((_TB, 1), lambda i: (i, 0)),
            pl.BlockSpec((_TB, 1), lambda i: (i, 0)),
            pl.BlockSpec((_TB, 1), lambda i: (i, 0)),
            pl.BlockSpec((_TB, 1), lambda i: (i, 0)),
        ),
        out_shape=(
            jax.ShapeDtypeStruct((_T, _D_MODEL), F32),
            jax.ShapeDtypeStruct((_T, 1), I32),
            jax.ShapeDtypeStruct((_T, 1), I32),
            jax.ShapeDtypeStruct((_T, 1), F32),
            jax.ShapeDtypeStruct((_T, 1), F32),
        ),
    )(x, og.reshape(1, -1), ob.reshape(1, -1), Wr)


def _kplan_body(e1_ref, e2_ref, dst_ref, eid_ref):
    lane1 = lax.broadcasted_iota(I32, (_T, _E), 1)
    x1 = (lane1 == e1_ref[...]).astype(F32)
    x2 = (lane1 == e2_ref[...]).astype(F32)
    cnt = (jnp.sum(x1, axis=0, keepdims=True)
           + jnp.sum(x2, axis=0, keepdims=True))
    padcnt = jnp.ceil(cnt / _SB) * _SB
    tri8 = (lax.broadcasted_iota(I32, (_E, _E), 0)
            < lax.broadcasted_iota(I32, (_E, _E), 1)).astype(F32)
    pad_off = _dot(padcnt, tri8)

    pbo = jnp.transpose(pad_off, (1, 0)) / _SB
    bidx = lax.broadcasted_iota(I32, (_E, _NBLK), 1).astype(F32)
    eid_ref[...] = (jnp.sum((bidx >= pbo).astype(I32), axis=0,
                            keepdims=True) - 1)

    tri = (lax.broadcasted_iota(I32, (_SB, _SB), 0)
           >= lax.broadcasted_iota(I32, (_SB, _SB), 1)).astype(F32)

    def block(b, x, carry):
        xb = x[b * _SB:(b + 1) * _SB, :]
        s = _dot(tri, xb)
        rank = s - xb + carry
        dst = jnp.sum(xb * (rank + pad_off), axis=1, keepdims=True)
        return dst.astype(I32), carry + s[_SB - 1:_SB, :]

    carry = jnp.zeros((1, _E), F32)
    for b in range(_T // _SB):
        dst, carry = block(b, x1, carry)
        dst_ref[pl.ds(b * _SB, _SB), :] = dst
    for b in range(_T // _SB):
        dst, carry = block(b, x2, carry)
        dst_ref[pl.ds(_T + b * _SB, _SB), :] = dst


def _plan(e1, e2):
    return pl.pallas_call(
        _kplan_body,
        grid=(1,),
        in_specs=[
            pl.BlockSpec((_T, 1), lambda i: (0, 0)),
            pl.BlockSpec((_T, 1), lambda i: (0, 0)),
        ],
        out_specs=(
            pl.BlockSpec((_NP, 1), lambda i: (0, 0)),
            pl.BlockSpec((1, _NBLK), lambda i: (0, 0)),
        ),
        out_shape=(
            jax.ShapeDtypeStruct((_NP, 1), I32),
            jax.ShapeDtypeStruct((1, _NBLK), I32),
        ),
    )(e1, e2)


def _sc_dispatch(xn, srctok, dst):
    info = plsc.get_sparse_core_info()
    nw = info.num_cores * info.num_subcores
    bpw = _NP // nw
    mesh = plsc.VectorSubcoreMesh(core_axis_name="c", subcore_axis_name="s")

    @functools.partial(
        pl.kernel, mesh=mesh,
        out_type=jax.ShapeDtypeStruct((_NSLOT, _D_MODEL), F32),
        scratch_types=[
            pltpu.VMEM((bpw,), I32),
            pltpu.VMEM((bpw,), I32),
            pltpu.VMEM((bpw, _D_MODEL), F32),
            pltpu.SemaphoreType.DMA,
            pltpu.SemaphoreType.DMA,
        ],
    )
    def k(xn_hbm, src_hbm, dst_hbm, xg_hbm, idxs_v, idxd_v, rows_v, s1, s2):
        wid = lax.axis_index("s") * info.num_cores + lax.axis_index("c")
        base = wid * bpw
        pltpu.sync_copy(src_hbm.at[pl.ds(base, bpw)], idxs_v)
        pltpu.sync_copy(dst_hbm.at[pl.ds(base, bpw)], idxd_v)
        pltpu.async_copy(xn_hbm.at[idxs_v], rows_v, s1).wait()
        pltpu.async_copy(rows_v, xg_hbm.at[idxd_v], s2).wait()

    return k(xn, srctok, dst)


def _sc_combine_gather(og, dst):
    info = plsc.get_sparse_core_info()
    nw = info.num_cores * info.num_subcores
    bpw = _NP // nw
    mesh = plsc.VectorSubcoreMesh(core_axis_name="c", subcore_axis_name="s")

    @functools.partial(
        pl.kernel, mesh=mesh,
        out_type=jax.ShapeDtypeStruct((_NP, _D_MODEL), F32),
        scratch_types=[
            pltpu.VMEM((bpw,), I32),
            pltpu.VMEM((bpw, _D_MODEL), F32),
            pltpu.SemaphoreType.DMA,
        ],
    )
    def k(og_hbm, dst_hbm, y2_hbm, idx_v, rows_v, sem):
        wid = lax.axis_index("s") * info.num_cores + lax.axis_index("c")
        base = wid * bpw
        pltpu.sync_copy(dst_hbm.at[pl.ds(base, bpw)], idx_v)
        pltpu.async_copy(og_hbm.at[idx_v], rows_v, sem).wait()
        pltpu.sync_copy(rows_v, y2_hbm.at[pl.ds(base, bpw)])

    return k(og, dst)


def _kgffn_body(eid_ref, xg_ref, w1_ref, b1_ref, w2_ref, b2_ref, og_ref):
    hpre = _dot(xg_ref[...], w1_ref[0]) + b1_ref[0]
    h = 0.5 * hpre * (1.0 + lax.erf(hpre * 0.7071067811865476))
    og_ref[...] = _dot(h, w2_ref[0]) + b2_ref[0]


def _grouped_ffn(eid, xg, W1, b1, W2, b2):
    grid_spec = pltpu.PrefetchScalarGridSpec(
        num_scalar_prefetch=1,
        grid=(_NBLK,),
        in_specs=[
            pl.BlockSpec((_SB, _D_MODEL), lambda b, eid_ref: (b, 0)),
            pl.BlockSpec((1, _D_MODEL, _DFF),
                         lambda b, eid_ref: (eid_ref[b], 0, 0)),
            pl.BlockSpec((1, 1, _DFF), lambda b, eid_ref: (eid_ref[b], 0, 0)),
            pl.BlockSpec((1, _DFF, _D_MODEL),
                         lambda b, eid_ref: (eid_ref[b], 0, 0)),
            pl.BlockSpec((1, 1, _D_MODEL),
                         lambda b, eid_ref: (eid_ref[b], 0, 0)),
        ],
        out_specs=pl.BlockSpec((_SB, _D_MODEL), lambda b, eid_ref: (b, 0)),
    )
    return pl.pallas_call(
        _kgffn_body,
        grid_spec=grid_spec,
        out_shape=jax.ShapeDtypeStruct((_NSLOT, _D_MODEL), F32),
    )(eid, xg, W1, b1.reshape(_E, 1, _DFF), W2, b2.reshape(_E, 1, _D_MODEL))


def _kcomb_body(r_ref, ya_ref, yb_ref, g1_ref, g2_ref, o_ref):
    o_ref[...] = (r_ref[...] + g1_ref[...] * ya_ref[...]
                  + g2_ref[...] * yb_ref[...])


def _combine(r, y2, g1, g2):
    nb = _T // _TB
    return pl.pallas_call(
        _kcomb_body,
        grid=(nb,),
        in_specs=[
            pl.BlockSpec((_TB, _D_MODEL), lambda i: (i, 0)),
            pl.BlockSpec((_TB, _D_MODEL), lambda i: (i, 0)),
            pl.BlockSpec((_TB, _D_MODEL), lambda i: (i + _T // _TB, 0)),
            pl.BlockSpec((_TB, 1), lambda i: (i, 0)),
            pl.BlockSpec((_TB, 1), lambda i: (i, 0)),
        ],
        out_specs=pl.BlockSpec((_TB, _D_MODEL), lambda i: (i, 0)),
        out_shape=jax.ShapeDtypeStruct((_T, _D_MODEL), F32),
    )(r, y2, y2, g1, g2)


def _moe_layer(x, og_, ob_, Wr, W1, b1, W2, b2, srctok):
    xn, e1, e2, g1, g2 = _router(x, og_, ob_, Wr)
    dst, eid = _plan(e1, e2)
    dst1 = dst.reshape(_NP)
    xg = _sc_dispatch(xn, srctok, dst1)
    og = _grouped_ffn(eid.reshape(_NBLK), xg, W1, b1, W2, b2)
    y2 = _sc_combine_gather(og, dst1)
    return _combine(x, y2, g1, g2)


def _kout_body(x_ref, w_ref, b_ref, g_ref, bb_ref, o_ref):
    o = _dot(x_ref[...], w_ref[...]) + b_ref[...]
    o_ref[...] = _ln_in(o, g_ref[...], bb_ref[...])


def _out_proj(x, W_out, b_out, ln_g, ln_b):
    nb = _T // _TB
    return pl.pallas_call(
        _kout_body,
        grid=(nb,),
        in_specs=[
            pl.BlockSpec((_TB, _D_MODEL), lambda i: (i, 0)),
            pl.BlockSpec((_D_MODEL, _D_OUT), lambda i: (0, 0)),
            pl.BlockSpec((1, _D_OUT), lambda i: (0, 0)),
            pl.BlockSpec((1, _D_OUT), lambda i: (0, 0)),
            pl.BlockSpec((1, _D_OUT), lambda i: (0, 0)),
        ],
        out_specs=pl.BlockSpec((_TB, _D_OUT), lambda i: (i, 0)),
        out_shape=jax.ShapeDtypeStruct((_T, _D_OUT), F32),
    )(x, W_out, b_out.reshape(1, -1), ln_g.reshape(1, -1),
      ln_b.reshape(1, -1))


def kernel(z_t, W_in, b_in, in_proj_W, conv_w, conv_b, x_proj_W, dt_proj_W,
           dt_proj_b, A_log, D_param, out_proj_W, router_W, eW1, eb1, eW2,
           eb2, mn_g, mn_b, on_g, on_b, W_out, b_out, ln_g, ln_b):
    z2d = z_t.reshape(_T, _D_IN)
    srctok = jnp.concatenate([jnp.arange(_T, dtype=I32)] * 2)
    x = _input_proj(z2d, W_in, b_in)
    for i in range(_NL):
        cwT = jnp.transpose(conv_w[i], (1, 0))
        negA_T = -jnp.exp(jnp.transpose(A_log[i], (1, 0)))
        xc, z, dt, bc = _mamba_front(x, mn_g[i], mn_b[i], in_proj_W[i], cwT,
                                     conv_b[i], x_proj_W[i], dt_proj_W[i],
                                     dt_proj_b[i])
        x = _mamba_scan(xc, z, dt, bc, negA_T, D_param[i], out_proj_W[i], x)
        x = _moe_layer(x, on_g[i], on_b[i], router_W[i], eW1[i], eb1[i],
                       eW2[i], eb2[i], srctok)
    out = _out_proj(x, W_out, b_out, ln_g, ln_b)
    return out.reshape(1, _T, _D_OUT)
